# baseline (speedup 1.0000x reference)
import functools

import jax
import jax.numpy as jnp
from jax import lax
from jax.experimental import pallas as pl
from jax.experimental.pallas import tpu as pltpu
from jax.experimental.pallas import tpu_sc as plsc


def _route_kernel(x_ref, w_ref,
                  mask1_ref, idx_ref, loc_ref, gmax_ref,
                  counts_ref, laux_ref,
                  base_ref, gsum_ref):
    i = pl.program_id(0)
    n = pl.num_programs(0)
    T = x_ref.shape[0]
    E = w_ref.shape[1]

    @pl.when(i == 0)
    def _init():
        base_ref[...] = jnp.zeros_like(base_ref)
        gsum_ref[...] = jnp.zeros_like(gsum_ref)

    x = x_ref[...]
    w = w_ref[...]
    logits = jnp.dot(x, w, preferred_element_type=jnp.float32)

    lmax = jnp.max(logits, axis=1, keepdims=True)
    ex = jnp.exp(logits - lmax)
    gates = ex / jnp.sum(ex, axis=1, keepdims=True)

    gmax = jnp.max(gates, axis=1, keepdims=True)
    eiota = lax.broadcasted_iota(jnp.int32, (T, E), 1)
    idx = jnp.min(jnp.where(gates == gmax, eiota, E), axis=1, keepdims=True)

    mask = (eiota == idx).astype(jnp.int32)
    mask_f = mask.astype(jnp.float32)
    row = lax.broadcasted_iota(jnp.int32, (T, T), 0)
    col = lax.broadcasted_iota(jnp.int32, (T, T), 1)
    tri = (col <= row).astype(jnp.float32)
    csum = jnp.dot(tri, mask_f, preferred_element_type=jnp.float32)
    base = base_ref[...].astype(jnp.float32)
    locs = csum - 1.0 + base
    loc = jnp.sum(locs * mask_f, axis=1, keepdims=True).astype(jnp.int32)

    new_base = base_ref[...] + jnp.sum(mask, axis=0, keepdims=True)
    base_ref[...] = new_base
    gsum_ref[...] = gsum_ref[...] + jnp.sum(gates, axis=0, keepdims=True)

    mask1_ref[...] = mask
    idx_ref[...] = idx
    loc_ref[...] = loc
    gmax_ref[...] = gmax
    counts_ref[...] = new_base
    S = n * T
    laux_ref[...] = jnp.sum(
        (gsum_ref[...] / S) * (new_base.astype(jnp.float32) / S),
        keepdims=True,
    ) * E


@functools.partial(jax.jit, static_argnames=("block_t",))
def _route(x, W, block_t=512):
    S, D = x.shape
    E = W.shape[1]
    n = S // block_t
    out_shapes = (
        jax.ShapeDtypeStruct((S, E), jnp.int32),
        jax.ShapeDtypeStruct((S, 1), jnp.int32),
        jax.ShapeDtypeStruct((S, 1), jnp.int32),
        jax.ShapeDtypeStruct((S, 1), jnp.float32),
        jax.ShapeDtypeStruct((1, E), jnp.int32),
        jax.ShapeDtypeStruct((1, 1), jnp.float32),
    )
    return pl.pallas_call(
        _route_kernel,
        grid=(n,),
        in_specs=[
            pl.BlockSpec((block_t, D), lambda i: (i, 0)),
            pl.BlockSpec((D, E), lambda i: (0, 0)),
        ],
        out_specs=(
            pl.BlockSpec((block_t, E), lambda i: (i, 0)),
            pl.BlockSpec((block_t, 1), lambda i: (i, 0)),
            pl.BlockSpec((block_t, 1), lambda i: (i, 0)),
            pl.BlockSpec((block_t, 1), lambda i: (i, 0)),
            pl.BlockSpec((1, E), lambda i: (0, 0)),
            pl.BlockSpec((1, 1), lambda i: (0, 0)),
        ),
        out_shape=out_shapes,
        scratch_shapes=[
            pltpu.VMEM((1, E), jnp.int32),
            pltpu.VMEM((1, E), jnp.float32),
        ],
    )(x, W)


def _combine_kernel(idx_ref, loc_ref, gmax_ref, combine_ref):
    T = idx_ref.shape[0]
    E = combine_ref.shape[1]
    C = combine_ref.shape[2]
    idx = idx_ref[...]
    loc = loc_ref[...]
    gmax = gmax_ref[...]
    eiota = lax.broadcasted_iota(jnp.int32, (T, E), 1)
    gate_val = jnp.where(eiota == idx, gmax, 0.0)
    ciota = lax.broadcasted_iota(jnp.int32, (T, C), 1)
    slot = (ciota == loc).astype(jnp.float32)
    combine_ref[...] = gate_val[:, :, None] * slot[:, None, :]


@functools.partial(jax.jit, static_argnames=("E", "C", "block_t"))
def _combine(idx, loc, gmax, E, C, block_t=512):
    S = idx.shape[0]
    n = S // block_t
    return pl.pallas_call(
        _combine_kernel,
        grid=(n,),
        in_specs=[
            pl.BlockSpec((block_t, 1), lambda i: (i, 0)),
            pl.BlockSpec((block_t, 1), lambda i: (i, 0)),
            pl.BlockSpec((block_t, 1), lambda i: (i, 0)),
        ],
        out_specs=pl.BlockSpec((block_t, E, C), lambda i: (i, 0, 0)),
        out_shape=jax.ShapeDtypeStruct((S, E, C), jnp.float32),
    )(idx, loc, gmax)


def _make_dispatch_sc(S, E, C):
    ROWS = S * E * C // 128
    info = plsc.get_sparse_core_info()
    NW = info.num_cores * info.num_subcores
    TPW = S // NW
    RPW = ROWS // NW
    ZROWS = 512
    NZ = RPW // ZROWS

    mesh = plsc.VectorSubcoreMesh(core_axis_name="c", subcore_axis_name="s")

    @functools.partial(
        pl.kernel,
        mesh=mesh,
        out_type=jax.ShapeDtypeStruct((ROWS, 128), jnp.bool_),
        scratch_types=[
            pltpu.VMEM((ZROWS, 128), jnp.bool_),
            pltpu.VMEM((TPW,), jnp.int32),
            pltpu.VMEM((TPW, 128), jnp.bool_),
            pltpu.SemaphoreType.DMA,
        ],
    )
    def dispatch_sc(wrow_hbm, pat_hbm, zero_hbm, out_hbm,
                    zbuf, idxbuf, patbuf, sem):
        wid = lax.axis_index("s") * info.num_cores + lax.axis_index("c")
        rbase = wid * RPW
        tbase = wid * TPW
        pltpu.sync_copy(zero_hbm, zbuf)
        pltpu.sync_copy(wrow_hbm.at[pl.ds(tbase, TPW)], idxbuf)
        pltpu.sync_copy(pat_hbm.at[pl.ds(tbase, TPW)], patbuf)
        for q in range(NZ):
            pltpu.sync_copy(zbuf, out_hbm.at[pl.ds(rbase + q * ZROWS, ZROWS)])
        pltpu.async_copy(patbuf, out_hbm.at[idxbuf], sem).wait()

    return dispatch_sc, ZROWS


def kernel(input, W):
    import numpy as np
    S, D = input.shape
    E = W.shape[1]
    C = max(int(np.ceil(S / E * 1.0)), 4)

    mask1, idx2, loc2, gmax2, counts, laux = _route(input, W)
    combine = _combine(idx2, loc2, gmax2, E, C)

    idx = idx2[:, 0]
    loc = loc2[:, 0]
    valid = (loc < C) & (gmax2[:, 0] != 0.0)
    wrow = jnp.arange(S, dtype=jnp.int32) * (E * C // 128) + jnp.where(
        valid, idx * (C // 128) + (loc >> 7), 0)
    pat = (jax.lax.broadcasted_iota(jnp.int32, (S, 128), 1)
           == (loc[:, None] & 127)) & valid[:, None]

    dispatch_sc, zrows = _make_dispatch_sc(S, E, C)
    zero = jnp.zeros((zrows, 128), dtype=jnp.bool_)
    dispatch = dispatch_sc(wrow, pat, zero).reshape(S, E, C)

    return (laux[0, 0], combine, dispatch, mask1, counts[0], idx)


# speedup vs baseline: 2.1688x; 2.1688x over previous
import functools

import jax
import jax.numpy as jnp
from jax import lax
from jax.experimental import pallas as pl
from jax.experimental.pallas import tpu as pltpu

_NBUF = 6


def _gate_kernel(x_ref, w_ref,
                 mask1_ref, idx_ref, loc_ref, gmax_ref,
                 counts_ref, laux_ref, combine_hbm,
                 bufs, sems, base_ref, gsum_ref):
    i = pl.program_id(0)
    n = pl.num_programs(0)
    T = x_ref.shape[0]
    E = w_ref.shape[1]
    C = bufs.shape[3]

    @pl.when(i == 0)
    def _init():
        base_ref[...] = jnp.zeros_like(base_ref)
        gsum_ref[...] = jnp.zeros_like(gsum_ref)

    x = x_ref[...]
    w = w_ref[...]
    logits = jnp.dot(x, w, preferred_element_type=jnp.float32)

    lmax = jnp.max(logits, axis=1, keepdims=True)
    ex = jnp.exp(logits - lmax)
    gates = ex / jnp.sum(ex, axis=1, keepdims=True)

    gmax = jnp.max(gates, axis=1, keepdims=True)
    eiota = lax.broadcasted_iota(jnp.int32, (T, E), 1)
    idx = jnp.min(jnp.where(gates == gmax, eiota, E), axis=1, keepdims=True)

    mask = (eiota == idx).astype(jnp.int32)
    mask_f = mask.astype(jnp.float32)
    row = lax.broadcasted_iota(jnp.int32, (T, T), 0)
    col = lax.broadcasted_iota(jnp.int32, (T, T), 1)
    tri = (col <= row).astype(jnp.float32)
    csum = jnp.dot(tri, mask_f, preferred_element_type=jnp.float32)
    base = base_ref[...].astype(jnp.float32)
    loc = jnp.sum((csum - 1.0 + base) * mask_f,
                  axis=1, keepdims=True).astype(jnp.int32)

    new_base = base_ref[...] + jnp.sum(mask, axis=0, keepdims=True)
    base_ref[...] = new_base
    gsum_ref[...] = gsum_ref[...] + jnp.sum(gates, axis=0, keepdims=True)

    mask1_ref[...] = mask
    idx_ref[...] = idx
    loc_ref[...] = loc
    gmax_ref[...] = gmax
    counts_ref[...] = new_base
    S = n * T
    laux_ref[...] = jnp.sum(
        (gsum_ref[...] / S) * (new_base.astype(jnp.float32) / S),
        keepdims=True,
    ) * E

    gate_val = jnp.where(eiota == idx, gmax, 0.0)
    ciota = lax.broadcasted_iota(jnp.int32, (T, C), 1)
    slot = (ciota == loc).astype(jnp.float32)
    sl = lax.rem(i, _NBUF)

    @pl.when(i >= _NBUF)
    def _drain():
        pltpu.make_async_copy(
            bufs.at[sl], combine_hbm.at[pl.ds((i - _NBUF) * T, T)], sems.at[sl]
        ).wait()

    bufs[sl] = gate_val[:, :, None] * slot[:, None, :]
    pltpu.make_async_copy(
        bufs.at[sl], combine_hbm.at[pl.ds(i * T, T)], sems.at[sl]
    ).start()

    @pl.when(i == n - 1)
    def _final():
        for k in range(_NBUF):
            step = n - _NBUF + k

            @pl.when(step >= 0)
            def _():
                s2 = lax.rem(jnp.int32(step), _NBUF)
                pltpu.make_async_copy(
                    bufs.at[s2], combine_hbm.at[pl.ds(step * T, T)], sems.at[s2]
                ).wait()


@functools.partial(jax.jit, static_argnames=("block_t",))
def _top1_gate(x, W, block_t=256):
    S, D = x.shape
    E = W.shape[1]
    import numpy as np
    C = max(int(np.ceil(S / E * 1.0)), 4)
    n = S // block_t

    out_shapes = (
        jax.ShapeDtypeStruct((S, E), jnp.int32),
        jax.ShapeDtypeStruct((S, 1), jnp.int32),
        jax.ShapeDtypeStruct((S, 1), jnp.int32),
        jax.ShapeDtypeStruct((S, 1), jnp.float32),
        jax.ShapeDtypeStruct((1, E), jnp.int32),
        jax.ShapeDtypeStruct((1, 1), jnp.float32),
        jax.ShapeDtypeStruct((S, E, C), jnp.float32),
    )
    return pl.pallas_call(
        _gate_kernel,
        grid=(n,),
        in_specs=[
            pl.BlockSpec((block_t, D), lambda i: (i, 0)),
            pl.BlockSpec((D, E), lambda i: (0, 0)),
        ],
        out_specs=(
            pl.BlockSpec((block_t, E), lambda i: (i, 0)),
            pl.BlockSpec((block_t, 1), lambda i: (i, 0)),
            pl.BlockSpec((block_t, 1), lambda i: (i, 0)),
            pl.BlockSpec((block_t, 1), lambda i: (i, 0)),
            pl.BlockSpec((1, E), lambda i: (0, 0)),
            pl.BlockSpec((1, 1), lambda i: (0, 0)),
            pl.BlockSpec(memory_space=pl.ANY),
        ),
        out_shape=out_shapes,
        scratch_shapes=[
            pltpu.VMEM((_NBUF, block_t, E, C), jnp.float32),
            pltpu.SemaphoreType.DMA((_NBUF,)),
            pltpu.VMEM((1, E), jnp.int32),
            pltpu.VMEM((1, E), jnp.float32),
        ],
    )(x, W)


def kernel(input, W):
    import numpy as np
    S, D = input.shape
    E = W.shape[1]
    C = max(int(np.ceil(S / E * 1.0)), 4)

    mask1, idx2, loc2, gmax2, counts, laux, combine = _top1_gate(input, W)

    idx = idx2[:, 0]
    loc = loc2[:, 0]
    valid = (loc < C) & (gmax2[:, 0] != 0.0)
    dispatch = ((mask1 != 0) & valid[:, None])[:, :, None] & (
        lax.broadcasted_iota(jnp.int32, (S, 1, C), 2) == loc[:, None, None])

    return (laux[0, 0], combine, dispatch, mask1, counts[0], idx)
